# baseline (speedup 1.0000x reference)
import jax
import jax.numpy as jnp
from jax import lax
from jax.experimental import pallas as pl
from jax.experimental.pallas import tpu as pltpu
from jax.experimental.pallas import tpu_sc as plsc

B, L, D = 4096, 200, 64
PAD_D = 128
VOCAB2 = 500000

_info = plsc.get_sparse_core_info()
NC, NS = _info.num_cores, _info.num_subcores
NW = NC * NS
BLK = B // NW
NL = 2
LANES = 16
NG = BLK // LANES


def _body(idx_hbm, tok_hbm, pos_hbm, out_hbm, idx_v, pos_v, psp_v, ih0, ih1,
          rows0, rows1, wb0, wb1, gsem0, gsem1, osem):
    wid = lax.axis_index("s") * NC + lax.axis_index("c")
    b0 = wid * BLK
    pltpu.sync_copy(idx_hbm.at[:, pl.ds(b0, BLK)], idx_v)
    pltpu.sync_copy(pos_hbm, pos_v)

    rows = (rows0, rows1)
    gsems = (gsem0, gsem1)
    ihs = (ih0, ih1)
    wbs = (wb0, wb1)
    lane = lax.iota(jnp.int32, LANES)
    row_ids = [lane + (g * LANES) for g in range(NG)]

    def start_gather(l, s):
        for g in range(NG):
            sl = pl.ds(g * LANES, LANES)
            ihs[s][sl] = lax.shift_right_logical(idx_v[l, sl], 1)
        pltpu.async_copy(tok_hbm.at[ihs[s]], rows[s], gsems[s])

    def wait_gather(s):
        pltpu.make_async_copy(tok_hbm.at[ihs[s]], rows[s], gsems[s]).wait()

    def start_out(c, wbuf):
        pltpu.async_copy(
            wbs[wbuf], out_hbm.at[pl.ds(c * NL, NL), :, pl.ds(b0, BLK)], osem)

    def wait_out(c, wbuf):
        pltpu.make_async_copy(
            wbs[wbuf], out_hbm.at[pl.ds(c * NL, NL), :, pl.ds(b0, BLK)],
            osem).wait()

    def transpose_add(l, s, lw, wbuf):
        buf = rows[s]
        wb = wbs[wbuf]

        for k in range(D // LANES):
            pv = pos_v[l, pl.ds(k * LANES, LANES)]
            for j in range(LANES):
                psp_v[k * LANES + j, :] = jnp.full((LANES,), pv[j])

        pgs = tuple(
            lax.shift_left(idx_v[l, pl.ds(g * LANES, LANES)] & 1, 6)
            for g in range(NG))

        def d_body(d, carry):
            p = psp_v[d, :]
            for g in range(NG):
                col = carry[g] + d
                vals = plsc.load_gather(buf, [row_ids[g], col])
                wb[lw, d, pl.ds(g * LANES, LANES)] = vals + p
            return carry

        lax.fori_loop(0, D, d_body, pgs)

    start_gather(0, 0)
    start_gather(1, 1)

    def group(g, carry):
        for par in range(2):
            c = 2 * g + par

            @pl.when(c >= 2)
            def _():
                wait_out(c - 2, par)

            for s in range(NL):
                l = c * NL + s
                wait_gather(s)
                transpose_add(l, s, s, par)

                @pl.when(l + 2 < L)
                def _():
                    start_gather(l + 2, s)

            start_out(c, par)
        return carry

    lax.fori_loop(0, L // NL // 2, group, None)
    wait_out(L // NL - 2, 0)
    wait_out(L // NL - 1, 1)


def kernel(inputs, token_table, pos_table):
    tok2 = jnp.reshape(token_table, (VOCAB2, PAD_D))
    out_t = pl.kernel(
        _body,
        out_type=jax.ShapeDtypeStruct((L, D, B), jnp.float32),
        mesh=plsc.VectorSubcoreMesh(core_axis_name="c", subcore_axis_name="s"),
        compiler_params=pltpu.CompilerParams(
            use_tc_tiling_on_sc=True, disable_bounds_checks=True,
            needs_layout_passes=False),
        scratch_types=[
            pltpu.VMEM((L, BLK), jnp.int32),
            pltpu.VMEM((L, D), jnp.float32),
            pltpu.VMEM((D, LANES), jnp.float32),
            pltpu.VMEM((BLK,), jnp.int32),
            pltpu.VMEM((BLK,), jnp.int32),
            pltpu.VMEM((BLK, PAD_D), jnp.float32),
            pltpu.VMEM((BLK, PAD_D), jnp.float32),
            pltpu.VMEM((NL, D, BLK), jnp.float32),
            pltpu.VMEM((NL, D, BLK), jnp.float32),
            pltpu.SemaphoreType.DMA,
            pltpu.SemaphoreType.DMA,
            pltpu.SemaphoreType.DMA,
        ],
    )(inputs.T, tok2, pos_table)
    return out_t.transpose(2, 0, 1)


# speedup vs baseline: 1.0034x; 1.0034x over previous
import jax
import jax.numpy as jnp
from jax import lax
from jax.experimental import pallas as pl
from jax.experimental.pallas import tpu as pltpu
from jax.experimental.pallas import tpu_sc as plsc

B, L, D = 4096, 200, 64

_info = plsc.get_sparse_core_info()
NC, NS = _info.num_cores, _info.num_subcores
NW = NC * NS
BLK = B // NW
NL = 2
LANES = 16
NG = BLK // LANES
ND8 = D // 8


def _body(idx_hbm, tok_hbm, pos_hbm, out_hbm, idx_v, pos_v, psp_v,
          rows0, rows1, wb0, wb1, gsem0, gsem1, osem):
    wid = lax.axis_index("s") * NC + lax.axis_index("c")
    b0 = wid * BLK
    pltpu.sync_copy(idx_hbm.at[:, pl.ds(b0, BLK)], idx_v)
    pltpu.sync_copy(pos_hbm, pos_v)

    rows = (rows0, rows1)
    gsems = (gsem0, gsem1)
    wbs = (wb0, wb1)
    lane = lax.iota(jnp.int32, LANES)
    row_ids = [lane + (g * LANES) for g in range(NG)]

    def start_gather(l, s):
        pltpu.async_copy(tok_hbm.at[idx_v.at[l]], rows[s], gsems[s])

    def wait_gather(l, s):
        pltpu.make_async_copy(tok_hbm.at[idx_v.at[l]], rows[s],
                              gsems[s]).wait()

    def start_out(c, wbuf):
        pltpu.async_copy(wbs[wbuf], out_hbm.at[pl.ds(c * NL, NL), :, wid],
                         osem)

    def wait_out(c, wbuf):
        pltpu.make_async_copy(wbs[wbuf],
                              out_hbm.at[pl.ds(c * NL, NL), :, wid],
                              osem).wait()

    def transpose_add(l, s, lw, wbuf):
        buf = rows[s]
        wb = wbs[wbuf]

        for k in range(D // LANES):
            pv = pos_v[l, pl.ds(k * LANES, LANES)]
            for j in range(LANES):
                psp_v[k * LANES + j, :] = jnp.full((LANES,), pv[j])

        def dd_body(dd, carry):
            base = jnp.full((LANES,), dd * 8, dtype=jnp.int32)
            for j in range(8):
                d = dd * 8 + j
                p = psp_v[d, :]
                col = base + j
                for g in range(NG):
                    vals = plsc.load_gather(buf, [row_ids[g], col])
                    wb[lw, dd, j, pl.ds(g * LANES, LANES)] = vals + p
            return carry

        lax.fori_loop(0, ND8, dd_body, None)

    start_gather(0, 0)
    start_gather(1, 1)

    def group(g, carry):
        for par in range(2):
            c = 2 * g + par

            @pl.when(c >= 2)
            def _():
                wait_out(c - 2, par)

            for s in range(NL):
                l = c * NL + s
                wait_gather(l, s)
                transpose_add(l, s, s, par)

                @pl.when(l + 2 < L)
                def _():
                    start_gather(l + 2, s)

            start_out(c, par)
        return carry

    lax.fori_loop(0, L // NL // 2, group, None)
    wait_out(L // NL - 2, 0)
    wait_out(L // NL - 1, 1)


def kernel(inputs, token_table, pos_table):
    out5 = pl.kernel(
        _body,
        out_type=jax.ShapeDtypeStruct((L, ND8, NW, 8, BLK), jnp.float32),
        mesh=plsc.VectorSubcoreMesh(core_axis_name="c", subcore_axis_name="s"),
        compiler_params=pltpu.CompilerParams(
            use_tc_tiling_on_sc=False, needs_layout_passes=False),
        scratch_types=[
            pltpu.VMEM((L, BLK), jnp.int32),
            pltpu.VMEM((L, D), jnp.float32),
            pltpu.VMEM((D, LANES), jnp.float32),
            pltpu.VMEM((BLK, D), jnp.float32),
            pltpu.VMEM((BLK, D), jnp.float32),
            pltpu.VMEM((NL, ND8, 8, BLK), jnp.float32),
            pltpu.VMEM((NL, ND8, 8, BLK), jnp.float32),
            pltpu.SemaphoreType.DMA,
            pltpu.SemaphoreType.DMA,
            pltpu.SemaphoreType.DMA,
        ],
    )(inputs.T, token_table, pos_table)
    return out5.transpose(2, 4, 0, 1, 3).reshape(B, L, D)
